# baseline (speedup 1.0000x reference)
import jax
import jax.numpy as jnp
from jax.experimental import pallas as pl
from jax.experimental.pallas import tpu as pltpu

_BM = 512


def _cdist_block(a_ref, b_ref, out_ref, b2_ref):
    @pl.when(pl.program_id(0) == 0)
    def _():
        bf = b_ref[...].astype(jnp.float32)
        b2_ref[...] = jnp.sum(bf * bf, axis=1)[None, :]

    af = a_ref[...].astype(jnp.float32)
    a2 = 0.25 * jnp.sum(af * af, axis=1, keepdims=True)
    mxu = jax.lax.dot_general(
        a_ref[...],
        b_ref[...],
        (((1,), (1,)), ((), ())),
        preferred_element_type=jnp.float32,
    )
    d2 = (a2 + b2_ref[...]) + mxu
    out_ref[...] = jnp.sqrt(jnp.maximum(d2, 0.0))


def kernel(source_mesh, target_mesh, state):
    del state
    m, k = source_mesh.shape
    n = target_mesh.shape[0]
    a_s = (-2.0 * source_mesh).astype(jnp.bfloat16)
    b_s = target_mesh.astype(jnp.bfloat16)
    return pl.pallas_call(
        _cdist_block,
        grid=(m // _BM,),
        in_specs=[
            pl.BlockSpec((_BM, k), lambda i: (i, 0)),
            pl.BlockSpec((n, k), lambda i: (0, 0)),
        ],
        out_specs=pl.BlockSpec((_BM, n), lambda i: (i, 0)),
        out_shape=jax.ShapeDtypeStruct((m, n), jnp.float32),
        scratch_shapes=[pltpu.VMEM((1, n), jnp.float32)],
    )(a_s, b_s)


# speedup vs baseline: 1.5145x; 1.5145x over previous
import jax
import jax.numpy as jnp
from jax.experimental import pallas as pl
from jax.experimental.pallas import tpu as pltpu

_BM = 512


def _cdist_block(a_ref, b_ref, out_ref, bbf_ref, b2_ref):
    @pl.when(pl.program_id(0) == 0)
    def _():
        bf = b_ref[...]
        bbf = bf.astype(jnp.bfloat16)
        bbf_ref[...] = bbf
        ones = jnp.ones((1, b_ref.shape[1]), jnp.bfloat16)
        b2_ref[...] = jax.lax.dot_general(
            ones,
            bbf * bbf,
            (((1,), (1,)), ((), ())),
            preferred_element_type=jnp.float32,
        )

    a = a_ref[...]
    a2 = jnp.sum(a * a, axis=1, keepdims=True)
    a_s = (-2.0 * a).astype(jnp.bfloat16)
    mxu = jax.lax.dot_general(
        a_s,
        bbf_ref[...],
        (((1,), (1,)), ((), ())),
        preferred_element_type=jnp.float32,
    )
    d2 = jnp.maximum((a2 + b2_ref[...]) + mxu, 1e-30)
    out_ref[...] = d2 * jax.lax.rsqrt(d2)


def kernel(source_mesh, target_mesh, state):
    del state
    m, k = source_mesh.shape
    n = target_mesh.shape[0]
    return pl.pallas_call(
        _cdist_block,
        grid=(m // _BM,),
        in_specs=[
            pl.BlockSpec((_BM, k), lambda i: (i, 0)),
            pl.BlockSpec((n, k), lambda i: (0, 0)),
        ],
        out_specs=pl.BlockSpec((_BM, n), lambda i: (i, 0)),
        out_shape=jax.ShapeDtypeStruct((m, n), jnp.float32),
        scratch_shapes=[
            pltpu.VMEM((n, k), jnp.bfloat16),
            pltpu.VMEM((1, n), jnp.float32),
        ],
    )(source_mesh, target_mesh)
